# baseline (speedup 1.0000x reference)
import functools

import jax
import jax.numpy as jnp
from jax import lax
from jax.experimental import pallas as pl
from jax.experimental.pallas import tpu as pltpu
from jax.experimental.pallas import tpu_sc as plsc

_HI = lax.Precision.HIGHEST

_NC = 2
_NS = 16
_NW = _NC * _NS
_B = 256
_BPW = _B // _NW


def _conv_s2d_body(x_ref, w_ref, b_ref, o_ref, *, hw):
    acc = None
    for dy in range(2):
        for dx in range(2):
            sl = x_ref[0, dy:dy + hw, dx:dx + hw, :]
            sl = sl.reshape(hw * hw, sl.shape[-1])
            t = jnp.dot(sl, w_ref[dy, dx], precision=_HI)
            acc = t if acc is None else acc + t
    o_ref[...] = jnp.maximum(acc + b_ref[...], 0.0)


def _conv_s2d(x, w, b, hw):
    co = w.shape[-1]
    c4 = x.shape[-1]
    return pl.pallas_call(
        functools.partial(_conv_s2d_body, hw=hw),
        grid=(8,),
        in_specs=[
            pl.BlockSpec((1, hw + 1, hw + 1, c4), lambda i: (i, 0, 0, 0)),
            pl.BlockSpec((2, 2, c4, co), lambda i: (0, 0, 0, 0)),
            pl.BlockSpec((1, co), lambda i: (0, 0)),
        ],
        out_specs=pl.BlockSpec((hw * hw, co), lambda i: (i, 0)),
        out_shape=jax.ShapeDtypeStruct((8 * hw * hw, co), jnp.float32),
    )(x, w, b.reshape(1, co))


def _s2d(x):
    if x.ndim == 3:
        x = x[..., None]
    b, h2, w2, c = x.shape
    return (x.reshape(b, h2 // 2, 2, w2 // 2, 2, c)
            .transpose(0, 1, 3, 2, 4, 5)
            .reshape(b, h2 // 2, w2 // 2, 4 * c))


def _w_s2d(w):
    kh, kw, ci, co = w.shape
    return (w.reshape(2, 2, 2, 2, ci, co)
            .transpose(0, 2, 1, 3, 4, 5)
            .reshape(2, 2, 4 * ci, co))


def _fc1_body(x_ref, w_ref, b_ref, o_ref):
    k = pl.program_id(0)

    @pl.when(k == 0)
    def _init():
        o_ref[...] = jnp.zeros_like(o_ref)

    o_ref[...] += jnp.dot(x_ref[...], w_ref[...], precision=_HI)

    @pl.when(k == pl.num_programs(0) - 1)
    def _fin():
        o_ref[...] = jnp.maximum(o_ref[...] + b_ref[...], 0.0)


def _fc1(x, w, b):
    return pl.pallas_call(
        _fc1_body,
        grid=(16,),
        in_specs=[
            pl.BlockSpec((8, 1024), lambda k: (0, k)),
            pl.BlockSpec((1024, 512), lambda k: (k, 0)),
            pl.BlockSpec((1, 512), lambda k: (0, 0)),
        ],
        out_specs=pl.BlockSpec((8, 512), lambda k: (0, 0)),
        out_shape=jax.ShapeDtypeStruct((8, 512), jnp.float32),
    )(x, w, b.reshape(1, 512))


def _mm_body(x_ref, w_ref, b_ref, o_ref, *, act):
    y = jnp.dot(x_ref[...], w_ref[...], precision=_HI) + b_ref[...]
    if act == "relu":
        y = jnp.maximum(y, 0.0)
    o_ref[...] = y


def _matmul(x, w, b, act):
    m, n = x.shape[0], w.shape[1]
    return pl.pallas_call(
        functools.partial(_mm_body, act=act),
        out_shape=jax.ShapeDtypeStruct((m, n), jnp.float32),
    )(x, w, b.reshape(1, n))


_VQ_CHUNK = 2048


def _vq_body(p_ref, e_ref, idx_ref, minv_ref):
    k = pl.program_id(0)

    @pl.when(k == 0)
    def _init():
        minv_ref[...] = jnp.full_like(minv_ref, jnp.inf)
        idx_ref[...] = jnp.zeros_like(idx_ref)

    p = p_ref[...]
    e = e_ref[...]
    p_aug = jnp.concatenate([-2.0 * p, jnp.ones((_B, 1), jnp.float32)], axis=1)
    e_aug = jnp.concatenate([e, jnp.sum(e * e, axis=1, keepdims=True)], axis=1)
    s = lax.dot_general(p_aug, e_aug, (((1,), (1,)), ((), ())), precision=_HI)
    m = jnp.min(s, axis=1, keepdims=True)
    col = lax.broadcasted_iota(jnp.int32, s.shape, 1)
    li = jnp.min(jnp.where(s <= m, col, jnp.int32(2**30)), axis=1, keepdims=True)
    better = m < minv_ref[...]
    idx_ref[...] = jnp.where(better, li + k * _VQ_CHUNK, idx_ref[...])
    minv_ref[...] = jnp.minimum(minv_ref[...], m)


def _vq(p, embeds):
    n = embeds.shape[0]
    return pl.pallas_call(
        _vq_body,
        grid=(n // _VQ_CHUNK,),
        in_specs=[
            pl.BlockSpec((_B, 32), lambda k: (0, 0)),
            pl.BlockSpec((_VQ_CHUNK, 32), lambda k: (k, 0)),
        ],
        out_specs=pl.BlockSpec((_B, 1), lambda k: (0, 0)),
        out_shape=jax.ShapeDtypeStruct((_B, 1), jnp.int32),
        scratch_shapes=[pltpu.VMEM((_B, 1), jnp.float32)],
    )(p, embeds)


@functools.cache
def _make_sc_gather():
    mesh = plsc.VectorSubcoreMesh(
        core_axis_name="c", subcore_axis_name="s", num_cores=_NC)

    @functools.partial(
        pl.kernel,
        mesh=mesh,
        out_type=jax.ShapeDtypeStruct((_B, 32), jnp.float32),
        scratch_types=[
            pltpu.VMEM((_BPW,), jnp.int32),
            pltpu.VMEM((_BPW, 32), jnp.float32),
            pltpu.SemaphoreType.DMA,
        ],
        compiler_params=pltpu.CompilerParams(use_tc_tiling_on_sc=False),
    )
    def _sc_gather(table_hbm, idx_hbm, out_hbm, idx_v, rows_v, sem):
        wid = lax.axis_index("s") * _NC + lax.axis_index("c")
        base = wid * _BPW
        pltpu.sync_copy(idx_hbm.at[pl.ds(base, _BPW)], idx_v)
        pltpu.async_copy(table_hbm.at[idx_v], rows_v, sem).wait()
        pltpu.sync_copy(rows_v, out_hbm.at[pl.ds(base, _BPW)])

    return _sc_gather


def _gather_rows(embeds, idx):
    return _make_sc_gather()(embeds, idx)


def _dec_body(pred_ref, col_ref, w1_ref, b1_ref, w2_ref, b2_ref, o_ref, y_ref):
    k = pl.program_id(0)

    @pl.when(k == 0)
    def _fc():
        ste = (-pred_ref[...] + col_ref[...]) + pred_ref[...]
        y_ref[...] = jnp.maximum(
            jnp.dot(ste, w1_ref[...], precision=_HI) + b1_ref[...], 0.0)

    y = y_ref[...]
    o_ref[...] = jnp.maximum(
        jnp.concatenate([jnp.dot(y, w2_ref[0, q], precision=_HI)
                         for q in range(8)], axis=1) + b2_ref[...], 0.0)


def _dec_fc(pred, col, dec_fc1_w, dec_fc1_b, w1_raw, bt1):
    return pl.pallas_call(
        _dec_body,
        grid=(8,),
        in_specs=[
            pl.BlockSpec((8, 1024), lambda k: (0, 0)),
            pl.BlockSpec((8, 1024), lambda k: (0, 0)),
            pl.BlockSpec((1024, 512), lambda k: (0, 0)),
            pl.BlockSpec((1, 512), lambda k: (0, 0)),
            pl.BlockSpec((1, 8, 512, 64), lambda k: (k, 0, 0, 0)),
            pl.BlockSpec((1, 512), lambda k: (0, k)),
        ],
        out_specs=pl.BlockSpec((8, 512), lambda k: (0, k)),
        out_shape=jax.ShapeDtypeStruct((8, 4096), jnp.float32),
        scratch_shapes=[pltpu.VMEM((8, 512), jnp.float32)],
    )(pred, col, dec_fc1_w, dec_fc1_b.reshape(1, 512), w1_raw,
      bt1.reshape(1, 4096))


def _dectail_body(y_ref, w2_ref, b2_ref, w3_ref, b3_ref, w4_ref, b4_ref, o_ref):
    y2 = jnp.maximum(jnp.dot(y_ref[...], w2_ref[...], precision=_HI)
                     + b2_ref[...], 0.0)
    y3 = jnp.concatenate(
        [jnp.maximum(jnp.dot(y2[:, 64 * g:64 * g + 64], w3_ref[...],
                             precision=_HI) + b3_ref[...], 0.0)
         for g in range(4)], axis=1)
    y4 = jnp.concatenate(
        [jnp.dot(y3[:, 32 * g:32 * g + 32], w4_ref[...], precision=_HI)
         + b4_ref[...]
         for g in range(16)], axis=1)
    o_ref[...] = jax.nn.sigmoid(y4)


def _dec_tail(y, wt2, bt2, wt3, bt3, wt4, bt4):
    return pl.pallas_call(
        _dectail_body,
        out_shape=jax.ShapeDtypeStruct((512, 64), jnp.float32),
    )(y, wt2, bt2.reshape(1, 256), wt3, bt3.reshape(1, 128),
      wt4, bt4.reshape(1, 4))


def _wt(w):
    kh, kw, ci, co = w.shape
    return w[::-1, ::-1].transpose(2, 0, 1, 3).reshape(ci, kh * kw * co)


def kernel(input_pl, enc_conv1_w, enc_conv1_b, enc_conv2_w, enc_conv2_b,
           enc_fc1_w, enc_fc1_b, pred_fc_w, pred_fc_b, embeds,
           dec_fc1_w, dec_fc1_b, dec_conv1_w, dec_conv1_b, dec_conv2_w,
           dec_conv2_b, dec_conv3_w, dec_conv3_b, dec_conv4_w, dec_conv4_b):
    x1 = _s2d(jnp.pad(input_pl, ((0, 0), (1, 1), (1, 1))))
    p1 = jnp.concatenate(
        [x1[:, dy:dy + 32, dx:dx + 32, :] for dy in range(2) for dx in range(2)],
        axis=-1)
    a1 = _matmul(p1.reshape(8192, 16), _w_s2d(enc_conv1_w).reshape(16, 32),
                 enc_conv1_b, "relu")

    x2 = _s2d(jnp.pad(a1.reshape(8, 32, 32, 32), ((0, 0), (1, 1), (1, 1), (0, 0))))
    a2 = _conv_s2d(x2, _w_s2d(enc_conv2_w), enc_conv2_b, 16)

    a3 = _fc1(a2.reshape(8, 16384), enc_fc1_w, enc_fc1_b)

    pred = _matmul(a3, pred_fc_w, pred_fc_b, "none")
    idx = _vq(pred.reshape(_B, 32), embeds)

    collected = _gather_rows(embeds, idx.reshape(_B))

    y = _dec_fc(pred, collected.reshape(8, 1024), dec_fc1_w, dec_fc1_b,
                dec_conv1_w, jnp.tile(dec_conv1_b, 64))
    y = y.reshape(512, 64)

    y = _dec_tail(y, _wt(dec_conv2_w), jnp.tile(dec_conv2_b, 4),
                  _wt(dec_conv3_w), jnp.tile(dec_conv3_b, 4),
                  _wt(dec_conv4_w), jnp.tile(dec_conv4_b, 4))

    return (y.reshape(8, 8, 8, 2, 2, 2, 2, 2, 2)[:, ::-1, ::-1]
            .transpose(0, 1, 3, 5, 7, 2, 4, 6, 8)
            .reshape(8, 64, 64, 1))


# speedup vs baseline: 1.0249x; 1.0249x over previous
import functools

import jax
import jax.numpy as jnp
from jax import lax
from jax.experimental import pallas as pl
from jax.experimental.pallas import tpu as pltpu
from jax.experimental.pallas import tpu_sc as plsc

_HI = lax.Precision.HIGHEST

_NC = 2
_NS = 16
_NW = _NC * _NS
_B = 256
_BPW = _B // _NW


def _conv_s2d_body(x_ref, w_ref, b_ref, o_ref, *, hw):
    acc = None
    for dy in range(2):
        for dx in range(2):
            sl = x_ref[0, dy:dy + hw, dx:dx + hw, :]
            sl = sl.reshape(hw * hw, sl.shape[-1])
            t = jnp.dot(sl, w_ref[dy, dx], precision=_HI)
            acc = t if acc is None else acc + t
    o_ref[...] = jnp.maximum(acc + b_ref[...], 0.0)


def _conv_s2d(x, w, b, hw):
    co = w.shape[-1]
    c4 = x.shape[-1]
    return pl.pallas_call(
        functools.partial(_conv_s2d_body, hw=hw),
        grid=(8,),
        in_specs=[
            pl.BlockSpec((1, hw + 1, hw + 1, c4), lambda i: (i, 0, 0, 0)),
            pl.BlockSpec((2, 2, c4, co), lambda i: (0, 0, 0, 0)),
            pl.BlockSpec((1, co), lambda i: (0, 0)),
        ],
        out_specs=pl.BlockSpec((hw * hw, co), lambda i: (i, 0)),
        out_shape=jax.ShapeDtypeStruct((8 * hw * hw, co), jnp.float32),
    )(x, w, b.reshape(1, co))


def _s2d(x):
    if x.ndim == 3:
        x = x[..., None]
    b, h2, w2, c = x.shape
    return (x.reshape(b, h2 // 2, 2, w2 // 2, 2, c)
            .transpose(0, 1, 3, 2, 4, 5)
            .reshape(b, h2 // 2, w2 // 2, 4 * c))


def _w_s2d(w):
    kh, kw, ci, co = w.shape
    return (w.reshape(2, 2, 2, 2, ci, co)
            .transpose(0, 2, 1, 3, 4, 5)
            .reshape(2, 2, 4 * ci, co))


def _fc1_body(x_ref, w_ref, b_ref, o_ref):
    k = pl.program_id(0)

    @pl.when(k == 0)
    def _init():
        o_ref[...] = jnp.zeros_like(o_ref)

    o_ref[...] += jnp.dot(x_ref[...], w_ref[...], precision=_HI)

    @pl.when(k == pl.num_programs(0) - 1)
    def _fin():
        o_ref[...] = jnp.maximum(o_ref[...] + b_ref[...], 0.0)


def _fc1(x, w, b):
    return pl.pallas_call(
        _fc1_body,
        grid=(8,),
        in_specs=[
            pl.BlockSpec((8, 2048), lambda k: (0, k)),
            pl.BlockSpec((2048, 512), lambda k: (k, 0)),
            pl.BlockSpec((1, 512), lambda k: (0, 0)),
        ],
        out_specs=pl.BlockSpec((8, 512), lambda k: (0, 0)),
        out_shape=jax.ShapeDtypeStruct((8, 512), jnp.float32),
    )(x, w, b.reshape(1, 512))


def _mm_body(x_ref, w_ref, b_ref, o_ref, *, act):
    y = jnp.dot(x_ref[...], w_ref[...], precision=_HI) + b_ref[...]
    if act == "relu":
        y = jnp.maximum(y, 0.0)
    o_ref[...] = y


def _matmul(x, w, b, act):
    m, n = x.shape[0], w.shape[1]
    return pl.pallas_call(
        functools.partial(_mm_body, act=act),
        out_shape=jax.ShapeDtypeStruct((m, n), jnp.float32),
    )(x, w, b.reshape(1, n))


_VQ_CHUNK = 2048


def _vq_body(p_ref, e_ref, idx_ref, minv_ref):
    k = pl.program_id(0)

    @pl.when(k == 0)
    def _init():
        minv_ref[...] = jnp.full_like(minv_ref, jnp.inf)
        idx_ref[...] = jnp.zeros_like(idx_ref)

    p = p_ref[...]
    e = e_ref[...]
    p_aug = jnp.concatenate([-2.0 * p, jnp.ones((_B, 1), jnp.float32)], axis=1)
    e_aug = jnp.concatenate([e, jnp.sum(e * e, axis=1, keepdims=True)], axis=1)
    s = lax.dot_general(p_aug, e_aug, (((1,), (1,)), ((), ())), precision=_HI)
    m = jnp.min(s, axis=1, keepdims=True)
    col = lax.broadcasted_iota(jnp.int32, s.shape, 1)
    li = jnp.min(jnp.where(s <= m, col, jnp.int32(2**30)), axis=1, keepdims=True)
    better = m < minv_ref[...]
    idx_ref[...] = jnp.where(better, li + k * _VQ_CHUNK, idx_ref[...])
    minv_ref[...] = jnp.minimum(minv_ref[...], m)


def _vq(p, embeds):
    n = embeds.shape[0]
    return pl.pallas_call(
        _vq_body,
        grid=(n // _VQ_CHUNK,),
        in_specs=[
            pl.BlockSpec((_B, 32), lambda k: (0, 0)),
            pl.BlockSpec((_VQ_CHUNK, 32), lambda k: (k, 0)),
        ],
        out_specs=pl.BlockSpec((_B, 1), lambda k: (0, 0)),
        out_shape=jax.ShapeDtypeStruct((_B, 1), jnp.int32),
        scratch_shapes=[pltpu.VMEM((_B, 1), jnp.float32)],
    )(p, embeds)


@functools.cache
def _make_sc_gather():
    mesh = plsc.VectorSubcoreMesh(
        core_axis_name="c", subcore_axis_name="s", num_cores=_NC)

    @functools.partial(
        pl.kernel,
        mesh=mesh,
        out_type=jax.ShapeDtypeStruct((_B, 32), jnp.float32),
        scratch_types=[
            pltpu.VMEM((_BPW,), jnp.int32),
            pltpu.VMEM((_BPW, 32), jnp.float32),
            pltpu.SemaphoreType.DMA,
        ],
        compiler_params=pltpu.CompilerParams(use_tc_tiling_on_sc=False),
    )
    def _sc_gather(table_hbm, idx_hbm, out_hbm, idx_v, rows_v, sem):
        wid = lax.axis_index("s") * _NC + lax.axis_index("c")
        base = wid * _BPW
        pltpu.sync_copy(idx_hbm.at[pl.ds(base, _BPW)], idx_v)
        pltpu.async_copy(table_hbm.at[idx_v], rows_v, sem).wait()
        pltpu.sync_copy(rows_v, out_hbm.at[pl.ds(base, _BPW)])

    return _sc_gather


def _gather_rows(embeds, idx):
    return _make_sc_gather()(embeds, idx)


def _dec_body(pred_ref, col_ref, w1_ref, b1_ref, w2_ref, b2_ref, o_ref, y_ref):
    k = pl.program_id(0)

    @pl.when(k == 0)
    def _fc():
        ste = (-pred_ref[...] + col_ref[...]) + pred_ref[...]
        y_ref[...] = jnp.maximum(
            jnp.dot(ste, w1_ref[...], precision=_HI) + b1_ref[...], 0.0)

    y = y_ref[...]
    o_ref[...] = jnp.maximum(
        jnp.concatenate([jnp.dot(y, w2_ref[0, q], precision=_HI)
                         for q in range(8)], axis=1) + b2_ref[...], 0.0)


def _dec_fc(pred, col, dec_fc1_w, dec_fc1_b, w1_raw, bt1):
    return pl.pallas_call(
        _dec_body,
        grid=(8,),
        in_specs=[
            pl.BlockSpec((8, 1024), lambda k: (0, 0)),
            pl.BlockSpec((8, 1024), lambda k: (0, 0)),
            pl.BlockSpec((1024, 512), lambda k: (0, 0)),
            pl.BlockSpec((1, 512), lambda k: (0, 0)),
            pl.BlockSpec((1, 8, 512, 64), lambda k: (k, 0, 0, 0)),
            pl.BlockSpec((1, 512), lambda k: (0, k)),
        ],
        out_specs=pl.BlockSpec((8, 512), lambda k: (0, k)),
        out_shape=jax.ShapeDtypeStruct((8, 4096), jnp.float32),
        scratch_shapes=[pltpu.VMEM((8, 512), jnp.float32)],
    )(pred, col, dec_fc1_w, dec_fc1_b.reshape(1, 512), w1_raw,
      bt1.reshape(1, 4096))


def _dectail_body(y_ref, w2_ref, b2_ref, w3_ref, b3_ref, w4_ref, b4_ref, o_ref):
    y2 = jnp.maximum(jnp.dot(y_ref[...], w2_ref[...], precision=_HI)
                     + b2_ref[...], 0.0)
    y3 = jnp.concatenate(
        [jnp.maximum(jnp.dot(y2[:, 64 * g:64 * g + 64], w3_ref[...],
                             precision=_HI) + b3_ref[...], 0.0)
         for g in range(4)], axis=1)
    y4 = jnp.concatenate(
        [jnp.dot(y3[:, 32 * g:32 * g + 32], w4_ref[...], precision=_HI)
         + b4_ref[...]
         for g in range(16)], axis=1)
    o_ref[...] = jax.nn.sigmoid(y4)


def _dec_tail(y, wt2, bt2, wt3, bt3, wt4, bt4):
    return pl.pallas_call(
        _dectail_body,
        out_shape=jax.ShapeDtypeStruct((512, 64), jnp.float32),
    )(y, wt2, bt2.reshape(1, 256), wt3, bt3.reshape(1, 128),
      wt4, bt4.reshape(1, 4))


def _wt(w):
    kh, kw, ci, co = w.shape
    return w[::-1, ::-1].transpose(2, 0, 1, 3).reshape(ci, kh * kw * co)


def kernel(input_pl, enc_conv1_w, enc_conv1_b, enc_conv2_w, enc_conv2_b,
           enc_fc1_w, enc_fc1_b, pred_fc_w, pred_fc_b, embeds,
           dec_fc1_w, dec_fc1_b, dec_conv1_w, dec_conv1_b, dec_conv2_w,
           dec_conv2_b, dec_conv3_w, dec_conv3_b, dec_conv4_w, dec_conv4_b):
    x1 = _s2d(jnp.pad(input_pl, ((0, 0), (1, 1), (1, 1))))
    p1 = jnp.concatenate(
        [x1[:, dy:dy + 32, dx:dx + 32, :] for dy in range(2) for dx in range(2)],
        axis=-1)
    a1 = _matmul(p1.reshape(8192, 16), _w_s2d(enc_conv1_w).reshape(16, 32),
                 enc_conv1_b, "relu")

    x2 = _s2d(jnp.pad(a1.reshape(8, 32, 32, 32), ((0, 0), (1, 1), (1, 1), (0, 0))))
    a2 = _conv_s2d(x2, _w_s2d(enc_conv2_w), enc_conv2_b, 16)

    a3 = _fc1(a2.reshape(8, 16384), enc_fc1_w, enc_fc1_b)

    pred = _matmul(a3, pred_fc_w, pred_fc_b, "none")
    idx = _vq(pred.reshape(_B, 32), embeds)

    collected = _gather_rows(embeds, idx.reshape(_B))

    y = _dec_fc(pred, collected.reshape(8, 1024), dec_fc1_w, dec_fc1_b,
                dec_conv1_w, jnp.tile(dec_conv1_b, 64))
    y = y.reshape(512, 64)

    y = _dec_tail(y, _wt(dec_conv2_w), jnp.tile(dec_conv2_b, 4),
                  _wt(dec_conv3_w), jnp.tile(dec_conv3_b, 4),
                  _wt(dec_conv4_w), jnp.tile(dec_conv4_b, 4))

    return (y.reshape(8, 8, 8, 2, 2, 2, 2, 2, 2)[:, ::-1, ::-1]
            .transpose(0, 1, 3, 5, 7, 2, 4, 6, 8)
            .reshape(8, 64, 64, 1))


# speedup vs baseline: 1.0427x; 1.0173x over previous
import functools

import jax
import jax.numpy as jnp
from jax import lax
from jax.experimental import pallas as pl
from jax.experimental.pallas import tpu as pltpu
from jax.experimental.pallas import tpu_sc as plsc

_HI = lax.Precision.HIGHEST

_NC = 2
_NS = 16
_NW = _NC * _NS
_B = 256
_BPW = _B // _NW


def _conv_s2d_body(x_ref, w_ref, b_ref, o_ref, *, hw):
    acc = None
    for dy in range(2):
        for dx in range(2):
            sl = x_ref[0, dy:dy + hw, dx:dx + hw, :]
            sl = sl.reshape(hw * hw, sl.shape[-1])
            t = jnp.dot(sl, w_ref[dy, dx], precision=_HI)
            acc = t if acc is None else acc + t
    o_ref[...] = jnp.maximum(acc + b_ref[...], 0.0)


def _conv_s2d(x, w, b, hw):
    co = w.shape[-1]
    c4 = x.shape[-1]
    return pl.pallas_call(
        functools.partial(_conv_s2d_body, hw=hw),
        grid=(8,),
        in_specs=[
            pl.BlockSpec((1, hw + 1, hw + 1, c4), lambda i: (i, 0, 0, 0)),
            pl.BlockSpec((2, 2, c4, co), lambda i: (0, 0, 0, 0)),
            pl.BlockSpec((1, co), lambda i: (0, 0)),
        ],
        out_specs=pl.BlockSpec((hw * hw, co), lambda i: (i, 0)),
        out_shape=jax.ShapeDtypeStruct((8 * hw * hw, co), jnp.float32),
    )(x, w, b.reshape(1, co))


def _s2d(x):
    if x.ndim == 3:
        x = x[..., None]
    b, h2, w2, c = x.shape
    return (x.reshape(b, h2 // 2, 2, w2 // 2, 2, c)
            .transpose(0, 1, 3, 2, 4, 5)
            .reshape(b, h2 // 2, w2 // 2, 4 * c))


def _w_s2d(w):
    kh, kw, ci, co = w.shape
    return (w.reshape(2, 2, 2, 2, ci, co)
            .transpose(0, 2, 1, 3, 4, 5)
            .reshape(2, 2, 4 * ci, co))


def _fc1_body(x_ref, w_ref, b_ref, pw_ref, pb_ref, o_ref, acc_ref):
    k = pl.program_id(0)

    @pl.when(k == 0)
    def _init():
        acc_ref[...] = jnp.zeros_like(acc_ref)

    acc_ref[...] += jnp.dot(x_ref[...], w_ref[...], precision=_HI)

    @pl.when(k == pl.num_programs(0) - 1)
    def _fin():
        a3 = jnp.maximum(acc_ref[...] + b_ref[...], 0.0)
        o_ref[...] = jnp.dot(a3, pw_ref[...], precision=_HI) + pb_ref[...]


def _fc1(x, w, b, pw, pb):
    return pl.pallas_call(
        _fc1_body,
        grid=(8,),
        in_specs=[
            pl.BlockSpec((8, 2048), lambda k: (0, k)),
            pl.BlockSpec((2048, 512), lambda k: (k, 0)),
            pl.BlockSpec((1, 512), lambda k: (0, 0)),
            pl.BlockSpec((512, 1024), lambda k: (0, 0)),
            pl.BlockSpec((1, 1024), lambda k: (0, 0)),
        ],
        out_specs=pl.BlockSpec((8, 1024), lambda k: (0, 0)),
        out_shape=jax.ShapeDtypeStruct((8, 1024), jnp.float32),
        scratch_shapes=[pltpu.VMEM((8, 512), jnp.float32)],
    )(x, w, b.reshape(1, 512), pw, pb.reshape(1, 1024))


def _mm_body(x_ref, w_ref, b_ref, o_ref, *, act):
    y = jnp.dot(x_ref[...], w_ref[...], precision=_HI) + b_ref[...]
    if act == "relu":
        y = jnp.maximum(y, 0.0)
    o_ref[...] = y


def _matmul(x, w, b, act):
    m, n = x.shape[0], w.shape[1]
    return pl.pallas_call(
        functools.partial(_mm_body, act=act),
        out_shape=jax.ShapeDtypeStruct((m, n), jnp.float32),
    )(x, w, b.reshape(1, n))


_VQ_CHUNK = 4096


def _vq_body(p_ref, e_ref, idx_ref, minv_ref):
    k = pl.program_id(0)

    @pl.when(k == 0)
    def _init():
        minv_ref[...] = jnp.full_like(minv_ref, jnp.inf)
        idx_ref[...] = jnp.zeros_like(idx_ref)

    p = p_ref[...]
    e = e_ref[...]
    p_aug = jnp.concatenate([-2.0 * p, jnp.ones((_B, 1), jnp.float32)], axis=1)
    e_aug = jnp.concatenate([e, jnp.sum(e * e, axis=1, keepdims=True)], axis=1)
    s = lax.dot_general(p_aug, e_aug, (((1,), (1,)), ((), ())), precision=_HI)
    m = jnp.min(s, axis=1, keepdims=True)
    col = lax.broadcasted_iota(jnp.int32, s.shape, 1)
    li = jnp.min(jnp.where(s <= m, col, jnp.int32(2**30)), axis=1, keepdims=True)
    better = m < minv_ref[...]
    idx_ref[...] = jnp.where(better, li + k * _VQ_CHUNK, idx_ref[...])
    minv_ref[...] = jnp.minimum(minv_ref[...], m)


def _vq(p, embeds):
    n = embeds.shape[0]
    return pl.pallas_call(
        _vq_body,
        grid=(n // _VQ_CHUNK,),
        in_specs=[
            pl.BlockSpec((_B, 32), lambda k: (0, 0)),
            pl.BlockSpec((_VQ_CHUNK, 32), lambda k: (k, 0)),
        ],
        out_specs=pl.BlockSpec((_B, 1), lambda k: (0, 0)),
        out_shape=jax.ShapeDtypeStruct((_B, 1), jnp.int32),
        scratch_shapes=[pltpu.VMEM((_B, 1), jnp.float32)],
    )(p, embeds)


@functools.cache
def _make_sc_gather():
    mesh = plsc.VectorSubcoreMesh(
        core_axis_name="c", subcore_axis_name="s", num_cores=_NC)

    @functools.partial(
        pl.kernel,
        mesh=mesh,
        out_type=jax.ShapeDtypeStruct((_B, 32), jnp.float32),
        scratch_types=[
            pltpu.VMEM((_BPW,), jnp.int32),
            pltpu.VMEM((_BPW, 32), jnp.float32),
            pltpu.SemaphoreType.DMA,
        ],
        compiler_params=pltpu.CompilerParams(use_tc_tiling_on_sc=False),
    )
    def _sc_gather(table_hbm, idx_hbm, out_hbm, idx_v, rows_v, sem):
        wid = lax.axis_index("s") * _NC + lax.axis_index("c")
        base = wid * _BPW
        pltpu.sync_copy(idx_hbm.at[pl.ds(base, _BPW)], idx_v)
        pltpu.async_copy(table_hbm.at[idx_v], rows_v, sem).wait()
        pltpu.sync_copy(rows_v, out_hbm.at[pl.ds(base, _BPW)])

    return _sc_gather


def _gather_rows(embeds, idx):
    return _make_sc_gather()(embeds, idx)


def _dec_body(pred_ref, col_ref, w1_ref, b1_ref, w2_ref, b2_ref, o_ref, y_ref):
    k = pl.program_id(0)

    @pl.when(k == 0)
    def _fc():
        ste = (-pred_ref[...] + col_ref[...]) + pred_ref[...]
        y_ref[...] = jnp.maximum(
            jnp.dot(ste, w1_ref[...], precision=_HI) + b1_ref[...], 0.0)

    y = y_ref[...]
    o_ref[...] = jnp.maximum(
        jnp.concatenate([jnp.dot(y, w2_ref[0, q], precision=_HI)
                         for q in range(8)], axis=1) + b2_ref[...], 0.0)


def _dec_fc(pred, col, dec_fc1_w, dec_fc1_b, w1_raw, bt1):
    return pl.pallas_call(
        _dec_body,
        grid=(8,),
        in_specs=[
            pl.BlockSpec((8, 1024), lambda k: (0, 0)),
            pl.BlockSpec((8, 1024), lambda k: (0, 0)),
            pl.BlockSpec((1024, 512), lambda k: (0, 0)),
            pl.BlockSpec((1, 512), lambda k: (0, 0)),
            pl.BlockSpec((1, 8, 512, 64), lambda k: (k, 0, 0, 0)),
            pl.BlockSpec((1, 512), lambda k: (0, k)),
        ],
        out_specs=pl.BlockSpec((8, 512), lambda k: (0, k)),
        out_shape=jax.ShapeDtypeStruct((8, 4096), jnp.float32),
        scratch_shapes=[pltpu.VMEM((8, 512), jnp.float32)],
    )(pred, col, dec_fc1_w, dec_fc1_b.reshape(1, 512), w1_raw,
      bt1.reshape(1, 4096))


def _dectail_body(y_ref, w2_ref, b2_ref, w3_ref, b3_ref, w4_ref, b4_ref, o_ref):
    y2 = jnp.maximum(jnp.dot(y_ref[...], w2_ref[...], precision=_HI)
                     + b2_ref[...], 0.0)
    y3 = jnp.concatenate(
        [jnp.maximum(jnp.dot(y2[:, 64 * g:64 * g + 64], w3_ref[...],
                             precision=_HI) + b3_ref[...], 0.0)
         for g in range(4)], axis=1)
    y4 = jnp.concatenate(
        [jnp.dot(y3[:, 32 * g:32 * g + 32], w4_ref[...], precision=_HI)
         + b4_ref[...]
         for g in range(16)], axis=1)
    o_ref[...] = jax.nn.sigmoid(y4)


def _dec_tail(y, wt2, bt2, wt3, bt3, wt4, bt4):
    return pl.pallas_call(
        _dectail_body,
        out_shape=jax.ShapeDtypeStruct((512, 64), jnp.float32),
    )(y, wt2, bt2.reshape(1, 256), wt3, bt3.reshape(1, 128),
      wt4, bt4.reshape(1, 4))


def _wt(w):
    kh, kw, ci, co = w.shape
    return w[::-1, ::-1].transpose(2, 0, 1, 3).reshape(ci, kh * kw * co)


def kernel(input_pl, enc_conv1_w, enc_conv1_b, enc_conv2_w, enc_conv2_b,
           enc_fc1_w, enc_fc1_b, pred_fc_w, pred_fc_b, embeds,
           dec_fc1_w, dec_fc1_b, dec_conv1_w, dec_conv1_b, dec_conv2_w,
           dec_conv2_b, dec_conv3_w, dec_conv3_b, dec_conv4_w, dec_conv4_b):
    x1 = _s2d(jnp.pad(input_pl, ((0, 0), (1, 1), (1, 1))))
    p1 = jnp.concatenate(
        [x1[:, dy:dy + 32, dx:dx + 32, :] for dy in range(2) for dx in range(2)],
        axis=-1)
    a1 = _matmul(p1.reshape(8192, 16), _w_s2d(enc_conv1_w).reshape(16, 32),
                 enc_conv1_b, "relu")

    x2 = _s2d(jnp.pad(a1.reshape(8, 32, 32, 32), ((0, 0), (1, 1), (1, 1), (0, 0))))
    a2 = _conv_s2d(x2, _w_s2d(enc_conv2_w), enc_conv2_b, 16)

    pred = _fc1(a2.reshape(8, 16384), enc_fc1_w, enc_fc1_b,
                pred_fc_w, pred_fc_b)

    idx = _vq(pred.reshape(_B, 32), embeds)

    collected = _gather_rows(embeds, idx.reshape(_B))

    y = _dec_fc(pred, collected.reshape(8, 1024), dec_fc1_w, dec_fc1_b,
                dec_conv1_w, jnp.tile(dec_conv1_b, 64))
    y = y.reshape(512, 64)

    y = _dec_tail(y, _wt(dec_conv2_w), jnp.tile(dec_conv2_b, 4),
                  _wt(dec_conv3_w), jnp.tile(dec_conv3_b, 4),
                  _wt(dec_conv4_w), jnp.tile(dec_conv4_b, 4))

    return (y.reshape(8, 8, 8, 2, 2, 2, 2, 2, 2)[:, ::-1, ::-1]
            .transpose(0, 1, 3, 5, 7, 2, 4, 6, 8)
            .reshape(8, 64, 64, 1))


# speedup vs baseline: 1.0623x; 1.0189x over previous
import functools

import jax
import jax.numpy as jnp
from jax import lax
from jax.experimental import pallas as pl
from jax.experimental.pallas import tpu as pltpu
from jax.experimental.pallas import tpu_sc as plsc

_HI = lax.Precision.HIGHEST

_NC = 2
_NS = 16
_NW = _NC * _NS
_B = 256
_BPW = _B // _NW


def _conv_s2d_body(x_ref, w_ref, b_ref, o_ref, *, hw):
    acc = None
    for dy in range(2):
        for dx in range(2):
            sl = x_ref[0, dy:dy + hw, dx:dx + hw, :]
            sl = sl.reshape(hw * hw, sl.shape[-1])
            t = jnp.dot(sl, w_ref[dy, dx], precision=_HI)
            acc = t if acc is None else acc + t
    o_ref[...] = jnp.maximum(acc + b_ref[...], 0.0)


def _conv_s2d(x, w, b, hw):
    co = w.shape[-1]
    c4 = x.shape[-1]
    return pl.pallas_call(
        functools.partial(_conv_s2d_body, hw=hw),
        grid=(8,),
        in_specs=[
            pl.BlockSpec((1, hw + 1, hw + 1, c4), lambda i: (i, 0, 0, 0)),
            pl.BlockSpec((2, 2, c4, co), lambda i: (0, 0, 0, 0)),
            pl.BlockSpec((1, co), lambda i: (0, 0)),
        ],
        out_specs=pl.BlockSpec((hw * hw, co), lambda i: (i, 0)),
        out_shape=jax.ShapeDtypeStruct((8 * hw * hw, co), jnp.float32),
    )(x, w, b.reshape(1, co))


def _s2d(x):
    if x.ndim == 3:
        x = x[..., None]
    b, h2, w2, c = x.shape
    return (x.reshape(b, h2 // 2, 2, w2 // 2, 2, c)
            .transpose(0, 1, 3, 2, 4, 5)
            .reshape(b, h2 // 2, w2 // 2, 4 * c))


def _w_s2d(w):
    kh, kw, ci, co = w.shape
    return (w.reshape(2, 2, 2, 2, ci, co)
            .transpose(0, 2, 1, 3, 4, 5)
            .reshape(2, 2, 4 * ci, co))


def _conv2_body(x_ref, w_ref, b_ref, o_ref):
    v = x_ref[0]
    rowv = {}
    for oy in (-1, 0, 1, 2):
        p = oy % 2
        base = (oy - p) // 2
        vp = v.reshape(16, 2, 32, 32)[:, p]
        if base == 1:
            vp = jnp.concatenate(
                [vp[1:], jnp.zeros((1, 32, 32), jnp.float32)], axis=0)
        elif base == -1:
            vp = jnp.concatenate(
                [jnp.zeros((1, 32, 32), jnp.float32), vp[:15]], axis=0)
        rowv[oy] = vp
    acc = None
    for ky in range(4):
        ry = rowv[ky - 1]
        for kx in range(4):
            ox = kx - 1
            q = ox % 2
            base = (ox - q) // 2
            u = ry.reshape(16, 16, 2, 32)[:, :, q, :]
            if base == 1:
                u = jnp.concatenate(
                    [u[:, 1:, :], jnp.zeros((16, 1, 32), jnp.float32)], axis=1)
            elif base == -1:
                u = jnp.concatenate(
                    [jnp.zeros((16, 1, 32), jnp.float32), u[:, :15, :]], axis=1)
            t = jnp.dot(u.reshape(256, 32), w_ref[ky, kx], precision=_HI)
            acc = t if acc is None else acc + t
    o_ref[...] = jnp.maximum(acc + b_ref[...], 0.0)


def _conv2(x, w, b):
    return pl.pallas_call(
        _conv2_body,
        grid=(8,),
        in_specs=[
            pl.BlockSpec((1, 32, 32, 32), lambda i: (i, 0, 0, 0)),
            pl.BlockSpec((4, 4, 32, 64), lambda i: (0, 0, 0, 0)),
            pl.BlockSpec((1, 64), lambda i: (0, 0)),
        ],
        out_specs=pl.BlockSpec((256, 64), lambda i: (i, 0)),
        out_shape=jax.ShapeDtypeStruct((2048, 64), jnp.float32),
    )(x, w, b.reshape(1, 64))


def _fc1_body(x_ref, w_ref, b_ref, pw_ref, pb_ref, o_ref, acc_ref):
    k = pl.program_id(0)

    @pl.when(k == 0)
    def _init():
        acc_ref[...] = jnp.zeros_like(acc_ref)

    acc_ref[...] += jnp.dot(x_ref[...], w_ref[...], precision=_HI)

    @pl.when(k == pl.num_programs(0) - 1)
    def _fin():
        a3 = jnp.maximum(acc_ref[...] + b_ref[...], 0.0)
        o_ref[...] = jnp.dot(a3, pw_ref[...], precision=_HI) + pb_ref[...]


def _fc1(x, w, b, pw, pb):
    return pl.pallas_call(
        _fc1_body,
        grid=(8,),
        in_specs=[
            pl.BlockSpec((8, 2048), lambda k: (0, k)),
            pl.BlockSpec((2048, 512), lambda k: (k, 0)),
            pl.BlockSpec((1, 512), lambda k: (0, 0)),
            pl.BlockSpec((512, 1024), lambda k: (0, 0)),
            pl.BlockSpec((1, 1024), lambda k: (0, 0)),
        ],
        out_specs=pl.BlockSpec((8, 1024), lambda k: (0, 0)),
        out_shape=jax.ShapeDtypeStruct((8, 1024), jnp.float32),
        scratch_shapes=[pltpu.VMEM((8, 512), jnp.float32)],
    )(x, w, b.reshape(1, 512), pw, pb.reshape(1, 1024))


def _mm_body(x_ref, w_ref, b_ref, o_ref, *, act):
    y = jnp.dot(x_ref[...], w_ref[...], precision=_HI) + b_ref[...]
    if act == "relu":
        y = jnp.maximum(y, 0.0)
    o_ref[...] = y


def _matmul(x, w, b, act):
    m, n = x.shape[0], w.shape[1]
    return pl.pallas_call(
        functools.partial(_mm_body, act=act),
        out_shape=jax.ShapeDtypeStruct((m, n), jnp.float32),
    )(x, w, b.reshape(1, n))


_VQ_CHUNK = 4096


def _vq_body(p_ref, e_ref, idx_ref, minv_ref):
    k = pl.program_id(0)

    @pl.when(k == 0)
    def _init():
        minv_ref[...] = jnp.full_like(minv_ref, jnp.inf)
        idx_ref[...] = jnp.zeros_like(idx_ref)

    p = p_ref[...]
    e = e_ref[...]
    p_aug = jnp.concatenate([-2.0 * p, jnp.ones((_B, 1), jnp.float32)], axis=1)
    e_aug = jnp.concatenate([e, jnp.sum(e * e, axis=1, keepdims=True)], axis=1)
    s = lax.dot_general(p_aug, e_aug, (((1,), (1,)), ((), ())), precision=_HI)
    m = jnp.min(s, axis=1, keepdims=True)
    col = lax.broadcasted_iota(jnp.int32, s.shape, 1)
    li = jnp.min(jnp.where(s <= m, col, jnp.int32(2**30)), axis=1, keepdims=True)
    better = m < minv_ref[...]
    idx_ref[...] = jnp.where(better, li + k * _VQ_CHUNK, idx_ref[...])
    minv_ref[...] = jnp.minimum(minv_ref[...], m)


def _vq(p, embeds):
    n = embeds.shape[0]
    return pl.pallas_call(
        _vq_body,
        grid=(n // _VQ_CHUNK,),
        in_specs=[
            pl.BlockSpec((_B, 32), lambda k: (0, 0)),
            pl.BlockSpec((_VQ_CHUNK, 32), lambda k: (k, 0)),
        ],
        out_specs=pl.BlockSpec((_B, 1), lambda k: (0, 0)),
        out_shape=jax.ShapeDtypeStruct((_B, 1), jnp.int32),
        scratch_shapes=[pltpu.VMEM((_B, 1), jnp.float32)],
    )(p, embeds)


@functools.cache
def _make_sc_gather():
    mesh = plsc.VectorSubcoreMesh(
        core_axis_name="c", subcore_axis_name="s", num_cores=_NC)

    @functools.partial(
        pl.kernel,
        mesh=mesh,
        out_type=jax.ShapeDtypeStruct((_B, 32), jnp.float32),
        scratch_types=[
            pltpu.VMEM((_BPW,), jnp.int32),
            pltpu.VMEM((_BPW, 32), jnp.float32),
            pltpu.SemaphoreType.DMA,
        ],
        compiler_params=pltpu.CompilerParams(use_tc_tiling_on_sc=False),
    )
    def _sc_gather(table_hbm, idx_hbm, out_hbm, idx_v, rows_v, sem):
        wid = lax.axis_index("s") * _NC + lax.axis_index("c")
        base = wid * _BPW
        pltpu.sync_copy(idx_hbm.at[pl.ds(base, _BPW)], idx_v)
        pltpu.async_copy(table_hbm.at[idx_v], rows_v, sem).wait()
        pltpu.sync_copy(rows_v, out_hbm.at[pl.ds(base, _BPW)])

    return _sc_gather


def _gather_rows(embeds, idx):
    return _make_sc_gather()(embeds, idx)


def _dec_body(pred_ref, col_ref, w1_ref, b1_ref, w2_ref, b2_ref, o_ref, y_ref):
    k = pl.program_id(0)

    @pl.when(k == 0)
    def _fc():
        ste = (-pred_ref[...] + col_ref[...]) + pred_ref[...]
        y_ref[...] = jnp.maximum(
            jnp.dot(ste, w1_ref[...], precision=_HI) + b1_ref[...], 0.0)

    y = y_ref[...]
    o_ref[...] = jnp.maximum(
        jnp.concatenate([jnp.dot(y, w2_ref[0, q], precision=_HI)
                         for q in range(8)], axis=1) + b2_ref[...], 0.0)


def _dec_fc(pred, col, dec_fc1_w, dec_fc1_b, w1_raw, bt1):
    return pl.pallas_call(
        _dec_body,
        grid=(8,),
        in_specs=[
            pl.BlockSpec((8, 1024), lambda k: (0, 0)),
            pl.BlockSpec((8, 1024), lambda k: (0, 0)),
            pl.BlockSpec((1024, 512), lambda k: (0, 0)),
            pl.BlockSpec((1, 512), lambda k: (0, 0)),
            pl.BlockSpec((1, 8, 512, 64), lambda k: (k, 0, 0, 0)),
            pl.BlockSpec((1, 512), lambda k: (0, k)),
        ],
        out_specs=pl.BlockSpec((8, 512), lambda k: (0, k)),
        out_shape=jax.ShapeDtypeStruct((8, 4096), jnp.float32),
        scratch_shapes=[pltpu.VMEM((8, 512), jnp.float32)],
    )(pred, col, dec_fc1_w, dec_fc1_b.reshape(1, 512), w1_raw,
      bt1.reshape(1, 4096))


def _dectail_body(y_ref, w2_ref, b2_ref, w3_ref, b3_ref, w4_ref, b4_ref, o_ref):
    y2 = jnp.maximum(jnp.dot(y_ref[...], w2_ref[...], precision=_HI)
                     + b2_ref[...], 0.0)
    y3 = jnp.concatenate(
        [jnp.maximum(jnp.dot(y2[:, 64 * g:64 * g + 64], w3_ref[...],
                             precision=_HI) + b3_ref[...], 0.0)
         for g in range(4)], axis=1)
    y4 = jnp.concatenate(
        [jnp.dot(y3[:, 32 * g:32 * g + 32], w4_ref[...], precision=_HI)
         + b4_ref[...]
         for g in range(16)], axis=1)
    o_ref[...] = jax.nn.sigmoid(y4)


def _dec_tail(y, wt2, bt2, wt3, bt3, wt4, bt4):
    return pl.pallas_call(
        _dectail_body,
        out_shape=jax.ShapeDtypeStruct((512, 64), jnp.float32),
    )(y, wt2, bt2.reshape(1, 256), wt3, bt3.reshape(1, 128),
      wt4, bt4.reshape(1, 4))


def _wt(w):
    kh, kw, ci, co = w.shape
    return w[::-1, ::-1].transpose(2, 0, 1, 3).reshape(ci, kh * kw * co)


def kernel(input_pl, enc_conv1_w, enc_conv1_b, enc_conv2_w, enc_conv2_b,
           enc_fc1_w, enc_fc1_b, pred_fc_w, pred_fc_b, embeds,
           dec_fc1_w, dec_fc1_b, dec_conv1_w, dec_conv1_b, dec_conv2_w,
           dec_conv2_b, dec_conv3_w, dec_conv3_b, dec_conv4_w, dec_conv4_b):
    x1 = _s2d(jnp.pad(input_pl, ((0, 0), (1, 1), (1, 1))))
    p1 = jnp.concatenate(
        [x1[:, dy:dy + 32, dx:dx + 32, :] for dy in range(2) for dx in range(2)],
        axis=-1)
    a1 = _matmul(p1.reshape(8192, 16), _w_s2d(enc_conv1_w).reshape(16, 32),
                 enc_conv1_b, "relu")

    a2 = _conv2(a1.reshape(8, 32, 32, 32), enc_conv2_w, enc_conv2_b)

    pred = _fc1(a2.reshape(8, 16384), enc_fc1_w, enc_fc1_b,
                pred_fc_w, pred_fc_b)

    idx = _vq(pred.reshape(_B, 32), embeds)

    collected = _gather_rows(embeds, idx.reshape(_B))

    y = _dec_fc(pred, collected.reshape(8, 1024), dec_fc1_w, dec_fc1_b,
                dec_conv1_w, jnp.tile(dec_conv1_b, 64))
    y = y.reshape(512, 64)

    y = _dec_tail(y, _wt(dec_conv2_w), jnp.tile(dec_conv2_b, 4),
                  _wt(dec_conv3_w), jnp.tile(dec_conv3_b, 4),
                  _wt(dec_conv4_w), jnp.tile(dec_conv4_b, 4))

    return (y.reshape(8, 8, 8, 2, 2, 2, 2, 2, 2)[:, ::-1, ::-1]
            .transpose(0, 1, 3, 5, 7, 2, 4, 6, 8)
            .reshape(8, 64, 64, 1))
